# baseline (speedup 1.0000x reference)
import jax
import jax.numpy as jnp
from jax.experimental import pallas as pl
from jax.experimental.pallas import tpu as pltpu

_MARGIN = 0.3
_N = 32
_F = 2048
_L = 64
_HALF = _N // 2
_SCALE = 1.0 / float(_F * _L)


_SC = 128


def _sort_chunk(x):
    li = jax.lax.broadcasted_iota(jnp.int32, x.shape, 1)
    for k in (2, 4, 8, 16, 32, 64):
        if k < 64:
            dir_up = (li & k) == 0
        j = k // 2
        while j >= 1:
            is_low = (li & j) == 0
            if k == 64:
                take_min = is_low
            else:
                take_min = dir_up == is_low
            rm = pltpu.roll(x, 128 - j, 1)
            rp = pltpu.roll(x, j, 1)
            partner = jnp.where(is_low, rm, rp)
            mn = jnp.minimum(x, partner)
            mx = jnp.maximum(x, partner)
            x = jnp.where(take_min, mn, mx)
            j //= 2
    return x


def _sort_body(x_ref, o_ref):
    for sc in range(x_ref.shape[1] // _SC):
        sl = slice(sc * _SC, (sc + 1) * _SC)
        o_ref[0, sl, :] = _sort_chunk(x_ref[0, sl, :])


def _dist_loss_body(s_ref, t_ref, tv_ref, o_ref):
    c = pl.program_id(0)
    i0 = c * _HALF
    tv = tv_ref[...]

    def loss_row(i, acc):
        row = s_ref[i0 + i]
        ti = t_ref[i0 + i]
        ap = jnp.full((1, 8), -jnp.inf, jnp.float32)
        an = jnp.full((1, 8), jnp.inf, jnp.float32)
        for jb in range(_N // 8):
            chunk = s_ref[jb * 8 : (jb + 1) * 8]
            d8 = jnp.sum(jnp.abs(chunk - row[None, :, :]), axis=(1, 2))
            d8 = (d8 * _SCALE).reshape(1, 8)
            m8 = tv[:, jb * 8 : (jb + 1) * 8] == ti
            ap = jnp.maximum(ap, jnp.where(m8, d8, -jnp.inf))
            an = jnp.minimum(an, jnp.where(m8, jnp.inf, d8))
        hp = jnp.max(ap)
        hn = jnp.min(an)
        return acc + jnp.maximum(hp - hn + _MARGIN, 0.0)

    total = jax.lax.fori_loop(0, _HALF, loss_row, jnp.float32(0.0))
    o_ref[...] = jnp.full((1, 8, 128), total, jnp.float32)


def kernel(inputs, targets):
    x = inputs.reshape(_N, _F * _L // 128, 128)
    cf = _F * _L // 128 // 1

    s = pl.pallas_call(
        _sort_body,
        grid=(2, _HALF),
        in_specs=[
            pl.BlockSpec((1, cf, 128), lambda c, i: (c * _HALF + i, 0, 0))
        ],
        out_specs=pl.BlockSpec((1, cf, 128), lambda c, i: (c * _HALF + i, 0, 0)),
        out_shape=jax.ShapeDtypeStruct((_N, cf, 128), jnp.float32),
        compiler_params=pltpu.CompilerParams(
            dimension_semantics=("arbitrary", "arbitrary"),
            vmem_limit_bytes=64 * 1024 * 1024,
        ),
    )(x)

    partial = pl.pallas_call(
        _dist_loss_body,
        grid=(2,),
        in_specs=[
            pl.BlockSpec((_N, cf, 128), lambda c: (0, 0, 0)),
            pl.BlockSpec(memory_space=pltpu.SMEM),
            pl.BlockSpec((1, _N), lambda c: (0, 0)),
        ],
        out_specs=pl.BlockSpec((1, 8, 128), lambda c: (c, 0, 0)),
        out_shape=jax.ShapeDtypeStruct((2, 8, 128), jnp.float32),
        compiler_params=pltpu.CompilerParams(
            dimension_semantics=("arbitrary",),
            vmem_limit_bytes=48 * 1024 * 1024,
        ),
    )(s, targets.astype(jnp.int32), targets.astype(jnp.int32).reshape(1, _N))

    return (partial[0, 0, 0] + partial[1, 0, 0]) / jnp.float32(_N)


# speedup vs baseline: 1.7453x; 1.7453x over previous
import jax
import jax.numpy as jnp
from jax.experimental import pallas as pl
from jax.experimental.pallas import tpu as pltpu

_MARGIN = 0.3
_N = 32
_F = 2048
_L = 64
_CF = _F * _L // 128
_SCALE = 1.0 / float(_F * _L)
_INF = float("inf")


def _sort_sample(x):
    li = jax.lax.broadcasted_iota(jnp.int32, x.shape, 2)
    for k in (2, 4, 8, 16, 32, 64):
        if k < 64:
            dir_up = (li & k) == 0
        j = k // 2
        while j >= 1:
            is_low = (li & j) == 0
            if k == 64:
                take_min = is_low
            else:
                take_min = dir_up == is_low
            rm = pltpu.roll(x, 128 - j, 2)
            rp = pltpu.roll(x, j, 2)
            partner = jnp.where(is_low, rm, rp)
            mn = jnp.minimum(x, partner)
            mx = jnp.maximum(x, partner)
            x = jnp.where(take_min, mn, mx)
            j //= 2
    return x


def _body(x_ref, t_ref, trow_ref, o_ref, s_all, ap_s, an_s):
    k = pl.program_id(0)
    m = k - 1
    mc = jnp.maximum(m, 0)
    li = jax.lax.broadcasted_iota(jnp.int32, (1, 128), 1)

    anchor = s_all[mc]
    nchunks = (m >> 3) + 1

    def chunk_body(jb, row):
        chunk = s_all[pl.ds(jb * 8, 8)]
        d8 = jnp.sum(jnp.abs(chunk - anchor[None, :, :]), axis=(1, 2))
        d8 = (d8 * _SCALE).reshape(1, 8)
        d128 = jnp.concatenate([d8, jnp.zeros((1, 120), jnp.float32)], axis=1)
        return row + pltpu.roll(d128, jb * 8, 1)

    row = jax.lax.fori_loop(0, nchunks, chunk_body, jnp.zeros((1, 128), jnp.float32))

    tm = t_ref[mc]
    lanes_valid = li <= m
    same = jnp.logical_and(trow_ref[...] == tm, lanes_valid)
    diff = jnp.logical_and(trow_ref[...] != tm, lanes_valid)
    ap_c = jnp.where(same, row, -_INF)
    an_c = jnp.where(diff, row, _INF)
    rmax = jnp.max(ap_c, axis=1, keepdims=True)
    rmin = jnp.min(an_c, axis=1, keepdims=True)
    at_m = li == m
    apv = jnp.where(k == 0, -_INF, ap_s[...])
    anv = jnp.where(k == 0, _INF, an_s[...])
    apv = jnp.maximum(jnp.maximum(apv, ap_c), jnp.where(at_m, rmax, -_INF))
    anv = jnp.minimum(jnp.minimum(anv, an_c), jnp.where(at_m, rmin, _INF))
    ap_s[...] = apv
    an_s[...] = anv

    lossv = jnp.maximum(apv - anv + _MARGIN, 0.0)
    lossv = jnp.where(li < _N, lossv, 0.0)
    total = jnp.sum(lossv, axis=1, keepdims=True) * (1.0 / _N)
    o_ref[...] = jnp.broadcast_to(total[None], (1, 8, 128))

    s_all[pl.ds(jnp.minimum(k, _N - 1), 1)] = _sort_sample(x_ref[...])


def kernel(inputs, targets):
    x = inputs.reshape(_N, _CF, 128)
    t32 = targets.astype(jnp.int32)
    trow = jnp.concatenate([t32, jnp.full((96,), -1, jnp.int32)]).reshape(1, 128)

    out = pl.pallas_call(
        _body,
        grid=(_N + 1,),
        in_specs=[
            pl.BlockSpec((1, _CF, 128), lambda k: (jnp.minimum(k, _N - 1), 0, 0)),
            pl.BlockSpec(memory_space=pltpu.SMEM),
            pl.BlockSpec((1, 128), lambda k: (0, 0)),
        ],
        out_specs=pl.BlockSpec((1, 8, 128), lambda k: (0, 0, 0)),
        out_shape=jax.ShapeDtypeStruct((1, 8, 128), jnp.float32),
        scratch_shapes=[
            pltpu.VMEM((_N, _CF, 128), jnp.float32),
            pltpu.VMEM((1, 128), jnp.float32),
            pltpu.VMEM((1, 128), jnp.float32),
        ],
        compiler_params=pltpu.CompilerParams(
            dimension_semantics=("arbitrary",),
            vmem_limit_bytes=48 * 1024 * 1024,
        ),
    )(x, t32, trow)

    return out[0, 0, 0]


# speedup vs baseline: 2.1865x; 1.2528x over previous
import jax
import jax.numpy as jnp
from jax.experimental import pallas as pl
from jax.experimental.pallas import tpu as pltpu

_MARGIN = 0.3
_N = 32
_F = 2048
_L = 64
_W = _F * 8
_SCALE = 1.0 / float(_F * _L)
_INF = float("inf")


def _sort_sample(x):
    si = jax.lax.broadcasted_iota(jnp.int32, x.shape, 1)
    li = jax.lax.broadcasted_iota(jnp.int32, x.shape, 2)
    for k in (2, 4, 8, 16, 32, 64):
        if k == 64:
            dir_up = None
        elif k < 8:
            dir_up = (si & k) == 0
        else:
            dir_up = (li & (k >> 3)) == 0
        j = k // 2
        while j >= 1:
            if j < 8:
                is_low = (si & j) == 0
                rm = pltpu.roll(x, 8 - j, 1)
                rp = pltpu.roll(x, j, 1)
            else:
                jl = j >> 3
                is_low = (li & jl) == 0
                rm = pltpu.roll(x, _W - jl, 2)
                rp = pltpu.roll(x, jl, 2)
            take_min = is_low if dir_up is None else dir_up == is_low
            partner = jnp.where(is_low, rm, rp)
            mn = jnp.minimum(x, partner)
            mx = jnp.maximum(x, partner)
            x = jnp.where(take_min, mn, mx)
            j //= 2
    return x


def _body(x_ref, t_ref, trow_ref, o_ref, s_all, ap_s, an_s):
    k = pl.program_id(0)
    m = k - 1
    mc = jnp.maximum(m, 0)
    li = jax.lax.broadcasted_iota(jnp.int32, (1, 128), 1)

    anchor = s_all[mc]
    nchunks = (m >> 3) + 1

    def chunk_body(jb, row):
        chunk = s_all[pl.ds(jb * 8, 8)]
        d8 = jnp.sum(jnp.abs(chunk - anchor[None, :, :]), axis=(1, 2))
        d8 = (d8 * _SCALE).reshape(1, 8)
        d128 = jnp.concatenate([d8, jnp.zeros((1, 120), jnp.float32)], axis=1)
        return row + pltpu.roll(d128, jb * 8, 1)

    row = jax.lax.fori_loop(0, nchunks, chunk_body, jnp.zeros((1, 128), jnp.float32))

    tm = t_ref[mc]
    lanes_valid = li <= m
    same = jnp.logical_and(trow_ref[...] == tm, lanes_valid)
    diff = jnp.logical_and(trow_ref[...] != tm, lanes_valid)
    ap_c = jnp.where(same, row, -_INF)
    an_c = jnp.where(diff, row, _INF)
    rmax = jnp.max(ap_c, axis=1, keepdims=True)
    rmin = jnp.min(an_c, axis=1, keepdims=True)
    at_m = li == m
    apv = jnp.where(k == 0, -_INF, ap_s[...])
    anv = jnp.where(k == 0, _INF, an_s[...])
    apv = jnp.maximum(jnp.maximum(apv, ap_c), jnp.where(at_m, rmax, -_INF))
    anv = jnp.minimum(jnp.minimum(anv, an_c), jnp.where(at_m, rmin, _INF))
    ap_s[...] = apv
    an_s[...] = anv

    lossv = jnp.maximum(apv - anv + _MARGIN, 0.0)
    lossv = jnp.where(li < _N, lossv, 0.0)
    total = jnp.sum(lossv, axis=1, keepdims=True) * (1.0 / _N)
    o_ref[...] = jnp.broadcast_to(total[None], (1, 8, 128))

    s_all[pl.ds(jnp.minimum(k, _N - 1), 1)] = _sort_sample(x_ref[...])


def kernel(inputs, targets):
    x = (
        inputs.reshape(_N, _F, 8, 8)
        .transpose(0, 2, 1, 3)
        .reshape(_N, 8, _W)
    )
    t32 = targets.astype(jnp.int32)
    trow = jnp.concatenate([t32, jnp.full((96,), -1, jnp.int32)]).reshape(1, 128)

    out = pl.pallas_call(
        _body,
        grid=(_N + 1,),
        in_specs=[
            pl.BlockSpec((1, 8, _W), lambda k: (jnp.minimum(k, _N - 1), 0, 0)),
            pl.BlockSpec(memory_space=pltpu.SMEM),
            pl.BlockSpec((1, 128), lambda k: (0, 0)),
        ],
        out_specs=pl.BlockSpec((1, 8, 128), lambda k: (0, 0, 0)),
        out_shape=jax.ShapeDtypeStruct((1, 8, 128), jnp.float32),
        scratch_shapes=[
            pltpu.VMEM((_N, 8, _W), jnp.float32),
            pltpu.VMEM((1, 128), jnp.float32),
            pltpu.VMEM((1, 128), jnp.float32),
        ],
        compiler_params=pltpu.CompilerParams(
            dimension_semantics=("arbitrary",),
            vmem_limit_bytes=48 * 1024 * 1024,
        ),
    )(x, t32, trow)

    return out[0, 0, 0]


# speedup vs baseline: 2.5931x; 1.1860x over previous
import jax
import jax.numpy as jnp
from jax.experimental import pallas as pl
from jax.experimental.pallas import tpu as pltpu

_MARGIN = 0.3
_N = 32
_F = 2048
_L = 64
_R = 1024
_SCALE = 1.0 / float(_F * _L)
_INF = float("inf")


def _pack_sample(xr):
    x = jnp.concatenate([xr[:, :_R, :], xr[:, _R:, :]], axis=2)
    x = x.reshape(128, 8, 128)
    si = jax.lax.broadcasted_iota(jnp.int32, x.shape, 1)
    li = jax.lax.broadcasted_iota(jnp.int32, x.shape, 2)
    sx = si ^ li
    for b in (1, 2, 4):
        swap = (sx & b) != 0
        pick_a = (si & b) == 0
        a = pltpu.roll(pltpu.roll(x, 8 - b, 1), b, 2)
        bb = pltpu.roll(pltpu.roll(x, b, 1), 128 - b, 2)
        x = jnp.where(swap, jnp.where(pick_a, a, bb), x)
    return x


def _sort_sample(x):
    si = jax.lax.broadcasted_iota(jnp.int32, x.shape, 1)
    li = jax.lax.broadcasted_iota(jnp.int32, x.shape, 2)
    for k in (2, 4, 8, 16, 32, 64):
        if k == 64:
            dir_up = None
        elif k < 8:
            dir_up = (si & k) == 0
        else:
            dir_up = (li & k) == 0
        j = k // 2
        while j >= 1:
            if j < 8:
                is_low = (si & j) == 0
                rm = pltpu.roll(x, 8 - j, 1)
                rp = pltpu.roll(x, j, 1)
            else:
                is_low = (li & j) == 0
                rm = pltpu.roll(x, 128 - j, 2)
                rp = pltpu.roll(x, j, 2)
            take_min = is_low if dir_up is None else dir_up == is_low
            partner = jnp.where(is_low, rm, rp)
            mn = jnp.minimum(x, partner)
            mx = jnp.maximum(x, partner)
            x = jnp.where(take_min, mn, mx)
            j //= 2
    return x


def _body(x_ref, t_ref, trow_ref, o_ref, s_all, ap_s, an_s):
    k = pl.program_id(0)
    m = k - 1
    mc = jnp.maximum(m, 0)
    li = jax.lax.broadcasted_iota(jnp.int32, (1, 128), 1)

    anchor = s_all[mc]
    nchunks = (m >> 3) + 1

    def chunk_body(jb, row):
        chunk = s_all[pl.ds(jb * 8, 8)]
        d8 = jnp.sum(jnp.abs(chunk - anchor[None]), axis=(1, 2, 3))
        d8 = (d8 * _SCALE).reshape(1, 8)
        d128 = jnp.concatenate([d8, jnp.zeros((1, 120), jnp.float32)], axis=1)
        return row + pltpu.roll(d128, jb * 8, 1)

    row = jax.lax.fori_loop(0, nchunks, chunk_body, jnp.zeros((1, 128), jnp.float32))

    tm = t_ref[mc]
    lanes_valid = li <= m
    same = jnp.logical_and(trow_ref[...] == tm, lanes_valid)
    diff = jnp.logical_and(trow_ref[...] != tm, lanes_valid)
    ap_c = jnp.where(same, row, -_INF)
    an_c = jnp.where(diff, row, _INF)
    rmax = jnp.max(ap_c, axis=1, keepdims=True)
    rmin = jnp.min(an_c, axis=1, keepdims=True)
    at_m = li == m
    apv = jnp.where(k == 0, -_INF, ap_s[...])
    anv = jnp.where(k == 0, _INF, an_s[...])
    apv = jnp.maximum(jnp.maximum(apv, ap_c), jnp.where(at_m, rmax, -_INF))
    anv = jnp.minimum(jnp.minimum(anv, an_c), jnp.where(at_m, rmin, _INF))
    ap_s[...] = apv
    an_s[...] = anv

    lossv = jnp.maximum(apv - anv + _MARGIN, 0.0)
    lossv = jnp.where(li < _N, lossv, 0.0)
    total = jnp.sum(lossv, axis=1, keepdims=True) * (1.0 / _N)
    o_ref[...] = jnp.broadcast_to(total[None], (1, 8, 128))

    kc = jnp.minimum(k, _N - 1)
    s_all[pl.ds(kc, 1)] = _sort_sample(_pack_sample(x_ref[...]))[None]


def kernel(inputs, targets):
    t32 = targets.astype(jnp.int32)
    trow = jnp.concatenate([t32, jnp.full((96,), -1, jnp.int32)]).reshape(1, 128)

    out = pl.pallas_call(
        _body,
        grid=(_N + 1,),
        in_specs=[
            pl.BlockSpec((1, _F, _L), lambda k: (jnp.minimum(k, _N - 1), 0, 0)),
            pl.BlockSpec(memory_space=pltpu.SMEM),
            pl.BlockSpec((1, 128), lambda k: (0, 0)),
        ],
        out_specs=pl.BlockSpec((1, 8, 128), lambda k: (0, 0, 0)),
        out_shape=jax.ShapeDtypeStruct((1, 8, 128), jnp.float32),
        scratch_shapes=[
            pltpu.VMEM((_N, 128, 8, 128), jnp.float32),
            pltpu.VMEM((1, 128), jnp.float32),
            pltpu.VMEM((1, 128), jnp.float32),
        ],
        compiler_params=pltpu.CompilerParams(
            dimension_semantics=("arbitrary",),
            vmem_limit_bytes=48 * 1024 * 1024,
        ),
    )(inputs, t32, trow)

    return out[0, 0, 0]


# speedup vs baseline: 2.5952x; 1.0008x over previous
import jax
import jax.numpy as jnp
from jax.experimental import pallas as pl
from jax.experimental.pallas import tpu as pltpu

_MARGIN = 0.3
_N = 32
_F = 2048
_L = 64
_R = 1024
_SCALE = 1.0 / float(_F * _L)
_INF = float("inf")


def _pack_sample(xr):
    x = jnp.concatenate([xr[:, :_R, :], xr[:, _R:, :]], axis=2)
    x = x.reshape(128, 8, 128)
    si = jax.lax.broadcasted_iota(jnp.int32, x.shape, 1)
    li = jax.lax.broadcasted_iota(jnp.int32, x.shape, 2)
    sx = si ^ li
    for b in (1, 2, 4):
        swap = (sx & b) != 0
        pick_a = (si & b) == 0
        a = pltpu.roll(pltpu.roll(x, 8 - b, 1), b, 2)
        bb = pltpu.roll(pltpu.roll(x, b, 1), 128 - b, 2)
        x = jnp.where(swap, jnp.where(pick_a, a, bb), x)
    return x


def _sort_sample(x):
    si = jax.lax.broadcasted_iota(jnp.int32, x.shape, 1)
    li = jax.lax.broadcasted_iota(jnp.int32, x.shape, 2)
    lfull = (li & 56) | (si & 7)
    for k in (2, 4, 8, 16, 32, 64):
        kb = k.bit_length() - 1
        j = k // 2
        while j >= 1:
            jb = j.bit_length() - 1
            if j < 8:
                rm = pltpu.roll(x, 8 - j, 1)
                rp = pltpu.roll(x, j, 1)
                jmask = j
            else:
                rm = pltpu.roll(x, 128 - j, 2)
                rp = pltpu.roll(x, j, 2)
                jmask = j
            is_low = (lfull & jmask) == 0
            if k == 64:
                take_min = is_low
            else:
                u = (lfull >> jb) ^ (lfull >> kb)
                take_min = (u & 1) == 0
            partner = jnp.where(is_low, rm, rp)
            mn = jnp.minimum(x, partner)
            mx = jnp.maximum(x, partner)
            x = jnp.where(take_min, mn, mx)
            j //= 2
    return x


def _body(x_ref, t_ref, trow_ref, o_ref, s_all, ap_s, an_s):
    k = pl.program_id(0)
    m = k - 1
    mc = jnp.maximum(m, 0)
    li = jax.lax.broadcasted_iota(jnp.int32, (1, 128), 1)

    anchor = s_all[mc]
    nchunks = (m >> 3) + 1

    def chunk_body(jb, row):
        chunk = s_all[pl.ds(jb * 8, 8)]
        d8 = jnp.sum(jnp.abs(chunk - anchor[None]), axis=(1, 2, 3))
        d8 = (d8 * _SCALE).reshape(1, 8)
        d128 = jnp.concatenate([d8, jnp.zeros((1, 120), jnp.float32)], axis=1)
        return row + pltpu.roll(d128, jb * 8, 1)

    row = jax.lax.fori_loop(0, nchunks, chunk_body, jnp.zeros((1, 128), jnp.float32))

    tm = t_ref[mc]
    lanes_valid = li <= m
    same = jnp.logical_and(trow_ref[...] == tm, lanes_valid)
    diff = jnp.logical_and(trow_ref[...] != tm, lanes_valid)
    ap_c = jnp.where(same, row, -_INF)
    an_c = jnp.where(diff, row, _INF)
    rmax = jnp.max(ap_c, axis=1, keepdims=True)
    rmin = jnp.min(an_c, axis=1, keepdims=True)
    at_m = li == m
    apv = jnp.where(k == 0, -_INF, ap_s[...])
    anv = jnp.where(k == 0, _INF, an_s[...])
    apv = jnp.maximum(jnp.maximum(apv, ap_c), jnp.where(at_m, rmax, -_INF))
    anv = jnp.minimum(jnp.minimum(anv, an_c), jnp.where(at_m, rmin, _INF))
    ap_s[...] = apv
    an_s[...] = anv

    lossv = jnp.maximum(apv - anv + _MARGIN, 0.0)
    lossv = jnp.where(li < _N, lossv, 0.0)
    total = jnp.sum(lossv, axis=1, keepdims=True) * (1.0 / _N)
    o_ref[...] = jnp.broadcast_to(total[None], (1, 8, 128))

    kc = jnp.minimum(k, _N - 1)
    s_all[pl.ds(kc, 1)] = _sort_sample(_pack_sample(x_ref[...]))[None]


def kernel(inputs, targets):
    t32 = targets.astype(jnp.int32)
    trow = jnp.concatenate([t32, jnp.full((96,), -1, jnp.int32)]).reshape(1, 128)

    out = pl.pallas_call(
        _body,
        grid=(_N + 1,),
        in_specs=[
            pl.BlockSpec((1, _F, _L), lambda k: (jnp.minimum(k, _N - 1), 0, 0)),
            pl.BlockSpec(memory_space=pltpu.SMEM),
            pl.BlockSpec((1, 128), lambda k: (0, 0)),
        ],
        out_specs=pl.BlockSpec((1, 8, 128), lambda k: (0, 0, 0)),
        out_shape=jax.ShapeDtypeStruct((1, 8, 128), jnp.float32),
        scratch_shapes=[
            pltpu.VMEM((_N, 128, 8, 128), jnp.float32),
            pltpu.VMEM((1, 128), jnp.float32),
            pltpu.VMEM((1, 128), jnp.float32),
        ],
        compiler_params=pltpu.CompilerParams(
            dimension_semantics=("arbitrary",),
            vmem_limit_bytes=48 * 1024 * 1024,
        ),
    )(inputs, t32, trow)

    return out[0, 0, 0]
